# Initial kernel scaffold; baseline (speedup 1.0000x reference)
import jax
import jax.numpy as jnp
from jax.experimental import pallas as pl


def kernel(in_feat, edge_index, W_self1, W_neigh1, b1, W_self2, W_neigh2, b2):
    raise NotImplementedError("write your pallas kernel here")


# speedup vs baseline: 4.9973x; 4.9973x over previous
import functools

import jax
import jax.numpy as jnp
from jax import lax
from jax.experimental import pallas as pl
from jax.experimental.pallas import tpu as pltpu
from jax.experimental.pallas import tpu_sc as plsc

_NC = 2
_NS = 16
_NW = _NC * _NS
_N = 10000
_E = 320000
_D = 128
_DEGW = 16
_C = 80
_PER_TILE = _E // _NW
_NCHUNK = _PER_TILE // _C
_RPT = 624
_TAIL_OFF = _RPT * _NS
_TAIL = _N - _TAIL_OFF

_mesh = plsc.VectorSubcoreMesh(core_axis_name="c", subcore_axis_name="s")


def _zero_acc(z128_hbm, acc_sh, sid):
    rb = sid * _RPT
    pltpu.sync_copy(z128_hbm.at[pl.ds(rb, _RPT)], acc_sh.at[pl.ds(rb, _RPT)])

    @pl.when(sid == 0)
    def _():
        pltpu.sync_copy(z128_hbm.at[pl.ds(_TAIL_OFF, _TAIL)],
                        acc_sh.at[pl.ds(_TAIL_OFF, _TAIL)])


def _drain_acc(acc_sh, out, cid, sid):
    rb = sid * _RPT
    pltpu.sync_copy(acc_sh.at[pl.ds(rb, _RPT)], out.at[cid, pl.ds(rb, _RPT)])

    @pl.when(sid == 0)
    def _():
        pltpu.sync_copy(acc_sh.at[pl.ds(_TAIL_OFF, _TAIL)],
                        out.at[cid, pl.ds(_TAIL_OFF, _TAIL)])


def _sc_agg_deg_body(src_hbm, dst_hbm, table_hbm, z128_hbm, ones_hbm,
                     acc_out, deg_out,
                     idx_s, idx_d, rows_v, ones_v, sem, acc_sh):
    cid = lax.axis_index("c")
    sid = lax.axis_index("s")
    wid = cid * _NS + sid
    base = wid * _PER_TILE

    _zero_acc(z128_hbm, acc_sh, sid)
    pltpu.sync_copy(ones_hbm, ones_v)
    plsc.subcore_barrier()

    def chunk_deg(j, carry):
        off = base + j * _C
        pltpu.sync_copy(dst_hbm.at[pl.ds(off, _C)], idx_d)
        pltpu.sync_copy(ones_v, acc_sh.at[idx_d], add=True)
        return carry

    lax.fori_loop(0, _NCHUNK, chunk_deg, 0)
    plsc.subcore_barrier()
    _drain_acc(acc_sh, deg_out, cid, sid)
    plsc.subcore_barrier()

    _zero_acc(z128_hbm, acc_sh, sid)
    plsc.subcore_barrier()

    def chunk(j, carry):
        off = base + j * _C
        pltpu.sync_copy(src_hbm.at[pl.ds(off, _C)], idx_s)
        pltpu.sync_copy(dst_hbm.at[pl.ds(off, _C)], idx_d)
        pltpu.async_copy(table_hbm.at[idx_s], rows_v, sem).wait()
        pltpu.sync_copy(rows_v, acc_sh.at[idx_d], add=True)
        return carry

    lax.fori_loop(0, _NCHUNK, chunk, 0)
    plsc.subcore_barrier()
    _drain_acc(acc_sh, acc_out, cid, sid)


def _sc_agg_body(src_hbm, dst_hbm, table_hbm, z128_hbm, acc_out,
                 idx_s, idx_d, rows_v, sem, acc_sh):
    cid = lax.axis_index("c")
    sid = lax.axis_index("s")
    wid = cid * _NS + sid
    base = wid * _PER_TILE

    _zero_acc(z128_hbm, acc_sh, sid)
    plsc.subcore_barrier()

    def chunk(j, carry):
        off = base + j * _C
        pltpu.sync_copy(src_hbm.at[pl.ds(off, _C)], idx_s)
        pltpu.sync_copy(dst_hbm.at[pl.ds(off, _C)], idx_d)
        pltpu.async_copy(table_hbm.at[idx_s], rows_v, sem).wait()
        pltpu.sync_copy(rows_v, acc_sh.at[idx_d], add=True)
        return carry

    lax.fori_loop(0, _NCHUNK, chunk, 0)
    plsc.subcore_barrier()
    _drain_acc(acc_sh, acc_out, cid, sid)


_sc_agg_deg = pl.kernel(
    _sc_agg_deg_body,
    mesh=_mesh,
    out_type=[
        jax.ShapeDtypeStruct((_NC, _N, _D), jnp.float32),
        jax.ShapeDtypeStruct((_NC, _N, _D), jnp.float32),
    ],
    scratch_types=[
        pltpu.VMEM((_C,), jnp.int32),
        pltpu.VMEM((_C,), jnp.int32),
        pltpu.VMEM((_C, _D), jnp.float32),
        pltpu.VMEM((_C, _D), jnp.float32),
        pltpu.SemaphoreType.DMA,
        pltpu.VMEM_SHARED((_N, _D), jnp.float32),
    ],
)

_sc_agg = pl.kernel(
    _sc_agg_body,
    mesh=_mesh,
    out_type=[
        jax.ShapeDtypeStruct((_NC, _N, _D), jnp.float32),
    ],
    scratch_types=[
        pltpu.VMEM((_C,), jnp.int32),
        pltpu.VMEM((_C,), jnp.int32),
        pltpu.VMEM((_C, _D), jnp.float32),
        pltpu.SemaphoreType.DMA,
        pltpu.VMEM_SHARED((_N, _D), jnp.float32),
    ],
)

_R = 1000


def _tc_mid_body(x_ref, a0_ref, a1_ref, d0_ref, d1_ref,
                 ws1_ref, wn1_ref, b1_ref, ws2_ref, wn2_ref, b2_ref,
                 p2_ref, hs2_ref):
    deg = jnp.maximum(d0_ref[:, 0:1] + d1_ref[:, 0:1], 1.0)
    hn1 = (a0_ref[...] + a1_ref[...]) / deg
    h1 = (jnp.dot(x_ref[...], ws1_ref[...], preferred_element_type=jnp.float32)
          + jnp.dot(hn1, wn1_ref[...], preferred_element_type=jnp.float32)
          + b1_ref[...])
    h1 = jnp.maximum(h1, 0.0)
    p2_ref[...] = jnp.dot(h1, wn2_ref[...], preferred_element_type=jnp.float32)
    hs2_ref[...] = (jnp.dot(h1, ws2_ref[...], preferred_element_type=jnp.float32)
                    + b2_ref[...])


def _tc_fin_body(hs2_ref, a0_ref, a1_ref, d0_ref, d1_ref, o_ref):
    deg = jnp.maximum(d0_ref[:, 0:1] + d1_ref[:, 0:1], 1.0)
    o_ref[...] = hs2_ref[...] + (a0_ref[...] + a1_ref[...]) / deg


def _row_spec(w):
    return pl.BlockSpec((_R, w), lambda i: (i, 0))


def _full_spec(h, w):
    return pl.BlockSpec((h, w), lambda i: (0, 0))


_tc_mid = pl.pallas_call(
    _tc_mid_body,
    grid=(_N // _R,),
    in_specs=[
        _row_spec(128), _row_spec(128), _row_spec(128),
        _row_spec(128), _row_spec(128),
        _full_spec(128, 256), _full_spec(128, 256), _full_spec(1, 256),
        _full_spec(256, 128), _full_spec(256, 128), _full_spec(1, 128),
    ],
    out_specs=[_row_spec(128), _row_spec(128)],
    out_shape=[
        jax.ShapeDtypeStruct((_N, 128), jnp.float32),
        jax.ShapeDtypeStruct((_N, 128), jnp.float32),
    ],
)

_tc_fin = pl.pallas_call(
    _tc_fin_body,
    grid=(_N // _R,),
    in_specs=[
        _row_spec(128), _row_spec(128), _row_spec(128),
        _row_spec(128), _row_spec(128),
    ],
    out_specs=_row_spec(128),
    out_shape=jax.ShapeDtypeStruct((_N, 128), jnp.float32),
)


def kernel(in_feat, edge_index, W_self1, W_neigh1, b1, W_self2, W_neigh2, b2):
    src = edge_index[0].astype(jnp.int32)
    dst = edge_index[1].astype(jnp.int32)
    z128 = jnp.zeros((_N, _D), jnp.float32)
    ones = jnp.ones((_C, _D), jnp.float32)

    acc1, degp = _sc_agg_deg(src, dst, in_feat, z128, ones)
    p2, hs2 = _tc_mid(in_feat, acc1[0], acc1[1], degp[0], degp[1],
                      W_self1, W_neigh1, b1.reshape(1, -1),
                      W_self2, W_neigh2, b2.reshape(1, -1))
    (acc2,) = _sc_agg(src, dst, p2, z128)
    return _tc_fin(hs2, acc2[0], acc2[1], degp[0], degp[1])
